# baseline (speedup 1.0000x reference)
import functools

import jax
import jax.numpy as jnp
from jax.experimental import pallas as pl
from jax.experimental.pallas import tpu as pltpu
from jax.experimental.pallas import tpu_sc as plsc

N = 2048
H = 4096
NNZ = 12288
BB = 128
D = 128
MD = 16
KNN = 16
EPS = 1e-5

_F32 = jnp.float32
_BF = jnp.bfloat16


def _ln2(h, g, b):
    mu = jnp.mean(h, axis=1, keepdims=True)
    var = jnp.mean((h - mu) ** 2, axis=1, keepdims=True)
    return (h - mu) / jnp.sqrt(var + EPS) * g + b


def _silu(v):
    return v / (1.0 + jnp.exp(-v))


def _full_spec(shape):
    nd = len(shape)
    return pl.BlockSpec(shape, lambda *_: (0,) * nd)


def _hl(w):
    wh = w.astype(_BF)
    wl = (w - wh.astype(_F32)).astype(_BF)
    return jnp.stack([wh, wl])


def _dothl(xf, w_ref):
    xh = xf.astype(_BF)
    xl = (xf - xh.astype(_F32)).astype(_BF)
    wh = w_ref[0]
    wl = w_ref[1]
    return (jnp.dot(xh, wh, preferred_element_type=_F32)
            + jnp.dot(xh, wl, preferred_element_type=_F32)
            + jnp.dot(xl, wh, preferred_element_type=_F32))


def _sc_gather(table, idx):
    m = idx.shape[1]
    c = table.shape[1]
    mesh = plsc.VectorSubcoreMesh(core_axis_name="c", subcore_axis_name="s")

    @functools.partial(
        pl.kernel,
        out_type=jax.ShapeDtypeStruct((m, c), table.dtype),
        mesh=mesh)
    def gk(tab_hbm, i_hbm, o_hbm):
        def body(i_vmem, o_vmem):
            pltpu.sync_copy(tab_hbm.at[i_vmem.at[0]], o_vmem)

        pltpu.emit_pipeline(
            body,
            grid=(m // 128,),
            in_specs=[pl.BlockSpec((1, 128), lambda i: (0, i))],
            out_specs=[pl.BlockSpec((128, c), lambda i: (i, 0))],
            core_axis_name=("c", "s"),
            dimension_semantics=(pltpu.PARALLEL,),
        )(i_hbm, o_hbm)

    return gk(table, idx)


def _tc_gather_body(x0_ref, nb_ref, out_ref):
    x0 = x0_ref[...]
    xh = x0.astype(_BF)
    xl = (x0 - xh.astype(_F32)).astype(_BF)
    nb = nb_ref[...]
    iot = jax.lax.broadcasted_iota(jnp.int32, (256, N), 1)
    for k in range(KNN):
        oh = (nb[:, k:k + 1] == iot).astype(_BF)
        out_ref[:, k, :] = (jnp.dot(oh, xh, preferred_element_type=_F32)
                            + jnp.dot(oh, xl, preferred_element_type=_F32))


def _tc_gather(x0, nb):
    return pl.pallas_call(
        _tc_gather_body,
        grid=(N // 256,),
        in_specs=[_full_spec((N, D)),
                  pl.BlockSpec((256, KNN), lambda i: (i, 0))],
        out_specs=pl.BlockSpec((256, KNN, D), lambda i: (i, 0, 0)),
        out_shape=jax.ShapeDtypeStruct((N, KNN, D), _F32),
    )(x0, nb)


def _embed_body(x_ref, ahi_ref, alo_ref, ea_ref, bond_ref, x0_ref, e0_ref):
    x = x_ref[...]
    iot = jax.lax.broadcasted_iota(jnp.int32, (N, 576), 1)
    oh = jnp.zeros((N, 576), _BF)
    for f in range(9):
        oh = oh + (x[:, f:f + 1] + f * 64 == iot).astype(_BF)
    x0_ref[...] = (jnp.dot(oh, ahi_ref[...], preferred_element_type=_F32)
                   + jnp.dot(oh, alo_ref[...], preferred_element_type=_F32))
    ea = ea_ref[...]
    e0 = jnp.zeros((H, D), _F32)
    for k in range(6):
        e0 = e0 + (ea == k).astype(_F32) * bond_ref[k:k + 1, :]
    e0_ref[...] = e0


def _embed(x, atom_hi, atom_lo, edge_attr, bond):
    return pl.pallas_call(
        _embed_body,
        out_shape=(jax.ShapeDtypeStruct((N, D), _F32),
                   jax.ShapeDtypeStruct((H, D), _F32)),
    )(x, atom_hi, atom_lo, edge_attr, bond)


_TT = 256


def _topk_body(pp_ref, pt_ref, dk_ref, nb_ref):
    pi = pp_ref[...]
    pt = pt_ref[...]
    d0 = pi[:, 0:1] - pt[0:1, :]
    d1 = pi[:, 1:2] - pt[1:2, :]
    d2 = pi[:, 2:3] - pt[2:3, :]
    dist = d0 * d0 + (d1 * d1 + d2 * d2)
    iot = jax.lax.broadcasted_iota(jnp.int32, (_TT, N), 1)
    kio = jax.lax.broadcasted_iota(jnp.int32, (_TT, KNN), 1)
    dk = jnp.zeros((_TT, KNN), _F32)
    nb = jnp.zeros((_TT, KNN), jnp.int32)
    for k in range(KNN):
        mn = jnp.min(dist, axis=1, keepdims=True)
        am = jnp.min(jnp.where(dist == mn, iot, N), axis=1, keepdims=True)
        dk = jnp.where(kio == k, mn, dk)
        nb = jnp.where(kio == k, am, nb)
        dist = jnp.where(iot == am, jnp.inf, dist)
    dk_ref[...] = dk
    nb_ref[...] = nb


def _topk(pos_pad, pos_t):
    return pl.pallas_call(
        _topk_body,
        grid=(N // _TT,),
        in_specs=[pl.BlockSpec((_TT, 8), lambda i: (i, 0)),
                  _full_spec((8, N))],
        out_specs=(pl.BlockSpec((_TT, KNN), lambda i: (i, 0)),
                   pl.BlockSpec((_TT, KNN), lambda i: (i, 0))),
        out_shape=(jax.ShapeDtypeStruct((N, KNN), _F32),
                   jax.ShapeDtypeStruct((N, KNN), jnp.int32)),
    )(pos_pad, pos_t)


_ET = 512


def _egnn_body(x0t_ref, fj_ref, dk_ref, w1i, w1j, w1d, b1,
               w2, b2, wn1a, wn1b, bn1, wn2, bn2, lng, lnb, out_ref):
    x0t = x0t_ref[...]
    dk = dk_ref[...]
    bi = _dothl(x0t, w1i) + b1[...]
    macc = jnp.zeros((_ET, MD), _F32)
    for k in range(KNN):
        fj = fj_ref[:, k, :]
        dcol = dk[:, k:k + 1]
        pre = bi + _dothl(fj, w1j) + dcol * w1d[...]
        hh = _silu(pre)
        m = _silu(_dothl(hh, w2) + b2[...])
        macc = macc + m * (dcol <= 25.0).astype(_F32)
    fln = _ln2(x0t, lng[...], lnb[...])
    h1 = _silu(_dothl(fln, wn1a) + _dothl(macc, wn1b) + bn1[...])
    out_ref[...] = x0t + _dothl(h1, wn2) + bn2[...]


def _egnn(x0, fj3, dk, ws):
    wspecs = [_full_spec(w.shape) for w in ws]
    return pl.pallas_call(
        _egnn_body,
        grid=(N // _ET,),
        in_specs=[pl.BlockSpec((_ET, D), lambda i: (i, 0)),
                  pl.BlockSpec((_ET, KNN, D), lambda i: (i, 0, 0)),
                  pl.BlockSpec((_ET, KNN), lambda i: (i, 0))] + wspecs,
        out_specs=pl.BlockSpec((_ET, D), lambda i: (i, 0)),
        out_shape=jax.ShapeDtypeStruct((N, D), _F32),
    )(x0, fj3, dk, *ws)


_TE = 128
_CH = 1024
_NCH = NNZ // _CH


def _inc_body(e3_ref, v3_ref, inc_ref):
    i = pl.program_id(0)
    j = pl.program_id(1)

    @pl.when(j == 0)
    def _zero():
        inc_ref[...] = jnp.zeros((_TE, N), _BF)

    ech = e3_ref[0]
    e0 = i * _TE
    emin = jnp.min(ech)
    emax = jnp.max(ech)

    @pl.when(jnp.logical_and(emax >= e0, emin < e0 + _TE))
    def _acc():
        vch = v3_ref[0]
        rows = jax.lax.broadcasted_iota(jnp.int32, (_TE, _CH), 0)
        a = (e0 + rows == ech).astype(_BF)
        cols = jax.lax.broadcasted_iota(jnp.int32, (_CH, N), 1)
        ohv = (vch == cols).astype(_BF)
        inc_ref[...] += jnp.dot(a, ohv,
                                preferred_element_type=_F32).astype(_BF)


def _inc(e3, v3):
    return pl.pallas_call(
        _inc_body,
        grid=(H // _TE, _NCH),
        in_specs=[pl.BlockSpec((1, 1, _CH), lambda i, j: (j, 0, 0)),
                  pl.BlockSpec((1, _CH, 1), lambda i, j: (j, 0, 0))],
        out_specs=pl.BlockSpec((_TE, N), lambda i, j: (i, 0)),
        out_shape=jax.ShapeDtypeStruct((H, N), _BF),
    )(e3, v3)


def _mv_body(x_ref, w0, b0, w1, b1, out_ref):
    x = x_ref[...]
    hh = jnp.maximum(_dothl(x, w0) + b0[...], 0.0)
    out_ref[...] = _dothl(hh, w1) + b1[...]


def _mv(x, ws):
    return pl.pallas_call(
        _mv_body,
        out_shape=jax.ShapeDtypeStruct((x.shape[0], D), _F32),
    )(x, *ws)


_TEB = 512


def _edge_body(inc_ref, mv_ref, e_ref, w20a, w20b, b20, w21, b21,
               lng, lnb, w30, b30, w31, b31, en_ref, me_ref, *, relu_out):
    inc = inc_ref[...]
    cnt = jnp.maximum(jnp.sum(inc.astype(_F32), axis=1, keepdims=True), 1.0)
    mv = mv_ref[...]
    mh = mv.astype(_BF)
    ml = (mv - mh.astype(_F32)).astype(_BF)
    agg = (jnp.dot(inc, mh, preferred_element_type=_F32)
           + jnp.dot(inc, ml, preferred_element_type=_F32)) / cnt
    e = e_ref[...]
    hh = jnp.maximum(_dothl(e, w20a) + _dothl(agg, w20b) + b20[...], 0.0)
    en = _ln2(_dothl(hh, w21) + b21[...], lng[...], lnb[...])
    h3 = jnp.maximum(_dothl(en, w30) + b30[...], 0.0)
    me_ref[...] = _dothl(h3, w31) + b31[...]
    en_ref[...] = jnp.maximum(en, 0.0) if relu_out else en


def _edge(inc, mv, e, ws, relu_out):
    wspecs = [_full_spec(w.shape) for w in ws]
    return pl.pallas_call(
        functools.partial(_edge_body, relu_out=relu_out),
        grid=(H // _TEB,),
        in_specs=[pl.BlockSpec((_TEB, N), lambda i: (i, 0)),
                  _full_spec((N, D)),
                  pl.BlockSpec((_TEB, D), lambda i: (i, 0))] + wspecs,
        out_specs=(pl.BlockSpec((_TEB, D), lambda i: (i, 0)),
                   pl.BlockSpec((_TEB, D), lambda i: (i, 0))),
        out_shape=(jax.ShapeDtypeStruct((H, D), _F32),
                   jax.ShapeDtypeStruct((H, D), _F32)),
    )(inc, mv, e, *ws)


_TVB = 512


def _node_body(incc_ref, me_ref, x_ref, w40a, w40b, b40, w41, b41,
               lng, lnb, out_ref, *, relu_out):
    incc = incc_ref[...]
    me = me_ref[...]
    mh = me.astype(_BF)
    ml = (me - mh.astype(_F32)).astype(_BF)
    dnum = (((0,), (0,)), ((), ()))
    agg = (jax.lax.dot_general(incc, mh, dnum, preferred_element_type=_F32)
           + jax.lax.dot_general(incc, ml, dnum, preferred_element_type=_F32))
    cnt = jax.lax.dot_general(incc, jnp.ones((H, 8), _BF), dnum,
                              preferred_element_type=_F32)[:, :1]
    agg = agg / jnp.maximum(cnt, 1.0)
    x = x_ref[...]
    hh = jnp.maximum(_dothl(x, w40a) + _dothl(agg, w40b) + b40[...], 0.0)
    xn = _ln2(_dothl(hh, w41) + b41[...], lng[...], lnb[...])
    out_ref[...] = jnp.maximum(xn, 0.0) if relu_out else xn


def _node(inc, me, x, ws, relu_out):
    wspecs = [_full_spec(w.shape) for w in ws]
    return pl.pallas_call(
        functools.partial(_node_body, relu_out=relu_out),
        grid=(N // _TVB,),
        in_specs=[pl.BlockSpec((H, _TVB), lambda i: (0, i)),
                  _full_spec((H, D)),
                  pl.BlockSpec((_TVB, D), lambda i: (i, 0))] + wspecs,
        out_specs=pl.BlockSpec((_TVB, D), lambda i: (i, 0)),
        out_shape=jax.ShapeDtypeStruct((N, D), _F32),
    )(inc, me, x, *ws)


def _head_body(x_ref, e_ref, bat_ref, eord_ref, o0a, o0b, b0, o1, b1o,
               out_ref):
    x = x_ref[...]
    e = e_ref[...]
    bat = bat_ref[...]
    ohb = (jax.lax.broadcasted_iota(jnp.int32, (BB, N), 0) == bat).astype(_BF)
    xh = x.astype(_BF)
    xl = (x - xh.astype(_F32)).astype(_BF)
    xp = (jnp.dot(ohb, xh, preferred_element_type=_F32)
          + jnp.dot(ohb, xl, preferred_element_type=_F32))
    eord = eord_ref[...]
    emask = (eord > 2).astype(_F32)
    r = jax.lax.broadcasted_iota(jnp.int32, (BB, H), 0)
    c = jax.lax.broadcasted_iota(jnp.int32, (BB, H), 1)
    oheb = (r == c // (H // BB)).astype(_BF)
    em = e * emask
    eh = em.astype(_BF)
    el = (em - eh.astype(_F32)).astype(_BF)
    ep = (jnp.dot(oheb, eh, preferred_element_type=_F32)
          + jnp.dot(oheb, el, preferred_element_type=_F32))
    hh = jnp.maximum(_dothl(xp, o0a) + _dothl(ep, o0b) + b0[...], 0.0)
    out_ref[...] = _dothl(hh, o1) + b1o[...]


def _head(x3, e3, bat, eord, ws):
    return pl.pallas_call(
        _head_body,
        out_shape=jax.ShapeDtypeStruct((BB, 1), _F32),
    )(x3, e3, bat, eord, *ws)


def kernel(x, pos, edge_index0, edge_index1, edge_attr, e_order, n_e,
           batch, params):
    p = params
    atom_flat = p['atom_emb'].reshape(9 * 64, D)
    atom_hi = atom_flat.astype(_BF)
    atom_lo = (atom_flat - atom_hi.astype(_F32)).astype(_BF)
    bond = p['bond_emb']

    x0, e0 = _embed(x, atom_hi, atom_lo, edge_attr, bond)

    pos_pad = jnp.pad(pos, ((0, 0), (0, 5)))
    pos_t = pos_pad.T
    dk, nb = _topk(pos_pad, pos_t)

    fj_flat = _sc_gather(x0, nb.reshape(1, N * KNN))
    fj3 = fj_flat.reshape(N, KNN, D)

    w1 = p['eg_e_w1']
    eg_ws = (_hl(w1[:D]), _hl(w1[D:2 * D]), w1[2 * D:2 * D + 1],
             p['eg_e_b1'].reshape(1, -1), _hl(p['eg_e_w2']),
             p['eg_e_b2'].reshape(1, -1),
             _hl(p['eg_n_w1'][:D]), _hl(p['eg_n_w1'][D:D + MD]),
             p['eg_n_b1'].reshape(1, -1), _hl(p['eg_n_w2']),
             p['eg_n_b2'].reshape(1, -1),
             p['eg_ln_g'].reshape(1, -1), p['eg_ln_b'].reshape(1, -1))
    xf = _egnn(x0, fj3, dk, eg_ws)

    e3 = edge_index1.reshape(_NCH, 1, _CH)
    v3 = edge_index0.reshape(_NCH, _CH, 1)
    inc = _inc(e3, v3)

    m1 = (_hl(p['m1_ws'][0]), p['m1_bs'][0].reshape(1, -1),
          _hl(p['m1_ws'][1]), p['m1_bs'][1].reshape(1, -1))
    m3 = (_hl(p['m3_ws'][0]), p['m3_bs'][0].reshape(1, -1),
          _hl(p['m3_ws'][1]), p['m3_bs'][1].reshape(1, -1))
    ew = (_hl(p['m2_ws'][0][:D]), _hl(p['m2_ws'][0][D:]),
          p['m2_bs'][0].reshape(1, -1), _hl(p['m2_ws'][1]),
          p['m2_bs'][1].reshape(1, -1),
          p['ln_e_g'].reshape(1, -1), p['ln_e_b'].reshape(1, -1)) + m3
    nw = (_hl(p['m4_ws'][0][:D]), _hl(p['m4_ws'][0][D:]),
          p['m4_bs'][0].reshape(1, -1), _hl(p['m4_ws'][1]),
          p['m4_bs'][1].reshape(1, -1),
          p['ln_x_g'].reshape(1, -1), p['ln_x_b'].reshape(1, -1))

    xc, ec = xf, e0
    for layer in range(3):
        relu_out = layer < 2
        mv = _mv(xc, m1)
        ec, me = _edge(inc, mv, ec, ew, relu_out)
        xc = _node(inc, me, xc, nw, relu_out)

    hw = (_hl(p['out_ws'][0][:D]), _hl(p['out_ws'][0][D:]),
          p['out_bs'][0].reshape(1, -1), _hl(p['out_ws'][1]),
          p['out_bs'][1].reshape(1, -1))
    out = _head(xc, ec, batch.reshape(1, N), e_order.reshape(H, 1), hw)
    return out.reshape(-1)
